# baseline (speedup 1.0000x reference)
import functools

import jax
import jax.numpy as jnp
from jax import lax
from jax.experimental import pallas as pl
from jax.experimental.pallas import tpu as pltpu
from jax.experimental.pallas import tpu_sc as plsc

N = 10000
E = 320000
D = 128

NC = 2
NS = 16
NW = NC * NS
L = 16

K = 80
CHUNKS = E // K
HALF_E = E // 2
HALF_CHUNKS = HALF_E // K
M_MAX = -(-HALF_CHUNKS // NW)
NBUF = 4
P = 2

N_PAD = 10240
SLAB = N_PAD // NS


def _sc_partial(edge_half, il3_half, zeros):
    mesh = plsc.VectorSubcoreMesh(core_axis_name="c", subcore_axis_name="s")

    @functools.partial(
        pl.kernel,
        out_type=jax.ShapeDtypeStruct((NC, N_PAD, D), jnp.float32),
        mesh=mesh,
        scratch_types=[
            pltpu.VMEM((NBUF, 2, K), jnp.int32),
            pltpu.VMEM((NBUF, K, D), jnp.float32),
            pltpu.VMEM_SHARED((N_PAD, D), jnp.float32),
            pltpu.SemaphoreType.DMA((NBUF,)),
            pltpu.SemaphoreType.DMA((NBUF,)),
            pltpu.SemaphoreType.DMA((NBUF,)),
        ],
        compiler_params=pltpu.CompilerParams(needs_layout_passes=False),
    )
    def k(edge_hbm, il_hbm, zeros_hbm, part_hbm,
          il_v, rows_v, acc, msem, rsem, ssem):
        c = lax.axis_index("c")
        s = lax.axis_index("s")
        wid = s * NC + c
        lane = lax.iota(jnp.int32, L)
        zero16 = jnp.zeros((L,), jnp.int32)

        pltpu.sync_copy(
            zeros_hbm.at[pl.ds(s * SLAB, SLAB)],
            acc.at[pl.ds(s * SLAB, SLAB)],
        )
        plsc.subcore_barrier()

        def issue_gathers(mm, b):
            cid = mm * NW + wid

            @pl.when(cid < HALF_CHUNKS)
            def _():
                pltpu.async_copy(il_hbm.at[cid], il_v.at[b], msem.at[b])
                pltpu.async_copy(
                    edge_hbm.at[pl.ds(cid * K, K), pl.ds(0, D)],
                    rows_v.at[b], rsem.at[b])

        def wait_meta(b):
            pltpu.make_async_copy(il_hbm.at[0], il_v.at[b], msem.at[b]).wait()

        def wait_rows(b):
            pltpu.make_async_copy(
                edge_hbm.at[pl.ds(0, K), pl.ds(0, D)],
                rows_v.at[b], rsem.at[b]).wait()

        def wait_scat(b):
            pltpu.make_async_copy(
                rows_v.at[b], acc.at[il_v.at[b, 0]], ssem.at[b]).wait()

        def consume(mm, b):
            cid = mm * NW + wid

            @pl.when(cid < HALF_CHUNKS)
            def _():
                wait_meta(b)
                wait_rows(b)
                for i in range(K // L):
                    vals = plsc.bitcast(
                        il_v[b, 1, pl.ds(i * L, L)], jnp.float32)
                    plsc.store_scatter(
                        rows_v.at[b], [lane + i * L, zero16], vals)
                pltpu.async_copy(
                    rows_v.at[b], acc.at[il_v.at[b, 0]], ssem.at[b],
                    add=True)

        for p in range(P):
            issue_gathers(p, p)

        ROUNDS = -(-M_MAX // NBUF)

        def body(r, carry):
            for b in range(NBUF):
                mm = r * NBUF + b
                consume(mm, b)
                kk = mm + P
                kb = (b + P) % NBUF
                kcid = kk * NW + wid

                @pl.when(kcid < HALF_CHUNKS)
                def _():
                    @pl.when(kk >= NBUF)
                    def _():
                        wait_scat(kb)
                    issue_gathers(kk, kb)
            return carry

        lax.fori_loop(0, ROUNDS, body, 0)

        for d in range(ROUNDS * NBUF - NBUF - P - 2, ROUNDS * NBUF):
            cid = d * NW + wid
            cid4 = (d + NBUF) * NW + wid

            @pl.when(jnp.logical_and(cid < HALF_CHUNKS,
                                     cid4 >= HALF_CHUNKS))
            def _():
                wait_scat(d % NBUF)

        plsc.subcore_barrier()

        pltpu.sync_copy(
            acc.at[pl.ds(s * SLAB, SLAB)],
            part_hbm.at[c, pl.ds(s * SLAB, SLAB)],
        )

    return k(edge_half, il3_half, zeros)


def _combine(vertex_attr, pa, pb):
    def body(v_ref, pa_ref, pb_ref, o_ref):
        p = pa_ref[0] + pa_ref[1] + pb_ref[0] + pb_ref[1]
        o_ref[:, :D] = v_ref[...]
        o_ref[:, D:] = jnp.concatenate([p[:, 1:], p[:, :1]], axis=1)

    return pl.pallas_call(
        body,
        grid=(10,),
        in_specs=[
            pl.BlockSpec((1000, D), lambda i: (i, 0)),
            pl.BlockSpec((NC, 1000, D), lambda i: (0, i, 0)),
            pl.BlockSpec((NC, 1000, D), lambda i: (0, i, 0)),
        ],
        out_specs=pl.BlockSpec((1000, 2 * D), lambda i: (i, 0)),
        out_shape=jax.ShapeDtypeStruct((N, 2 * D), jnp.float32),
    )(vertex_attr, pa, pb)


def kernel(vertex_attr, edgeij_pair, edge_attr, g, batch):
    dst2 = edgeij_pair[1].reshape(CHUNKS, 1, K)
    last2 = lax.bitcast_convert_type(
        edge_attr[:, D].reshape(CHUNKS, 1, K), jnp.int32)
    il3 = jnp.concatenate([dst2, last2], axis=1)
    zeros = jnp.zeros((N_PAD, D), dtype=jnp.float32)
    pa = _sc_partial(edge_attr[:HALF_E], il3[:HALF_CHUNKS], zeros)
    pb = _sc_partial(edge_attr[HALF_E:], il3[HALF_CHUNKS:], zeros)
    return _combine(vertex_attr, pa, pb)


# speedup vs baseline: 1.1928x; 1.1928x over previous
import functools

import jax
import jax.numpy as jnp
from jax import lax
from jax.experimental import pallas as pl
from jax.experimental.pallas import tpu as pltpu
from jax.experimental.pallas import tpu_sc as plsc

N = 10000
E = 320000
D = 128

NC = 2
NS = 16
NW = NC * NS
L = 16

K = 80
CHUNKS = E // K
HALF_E = E
HALF_CHUNKS = HALF_E // K
M_MAX = -(-HALF_CHUNKS // NW)
NBUF = 4
P = 2

N_PAD = 10240
SLAB = N_PAD // NS


def _sc_partial(edge_half, il3_half, zeros):
    mesh = plsc.VectorSubcoreMesh(core_axis_name="c", subcore_axis_name="s")

    @functools.partial(
        pl.kernel,
        out_type=jax.ShapeDtypeStruct((NC, N_PAD, D), jnp.float32),
        mesh=mesh,
        scratch_types=[
            pltpu.VMEM((NBUF, 2, K), jnp.int32),
            pltpu.VMEM((NBUF, K, D), jnp.float32),
            pltpu.VMEM_SHARED((N_PAD, D), jnp.float32),
            pltpu.SemaphoreType.DMA((NBUF,)),
            pltpu.SemaphoreType.DMA((NBUF,)),
            pltpu.SemaphoreType.DMA((NBUF,)),
        ],
        compiler_params=pltpu.CompilerParams(needs_layout_passes=False),
    )
    def k(edge_hbm, il_hbm, zeros_hbm, part_hbm,
          il_v, rows_v, acc, msem, rsem, ssem):
        c = lax.axis_index("c")
        s = lax.axis_index("s")
        wid = s * NC + c
        lane = lax.iota(jnp.int32, L)
        zero16 = jnp.zeros((L,), jnp.int32)

        pltpu.sync_copy(
            zeros_hbm.at[pl.ds(s * SLAB, SLAB)],
            acc.at[pl.ds(s * SLAB, SLAB)],
        )
        plsc.subcore_barrier()

        def issue_gathers(mm, b):
            cid = mm * NW + wid

            @pl.when(cid < HALF_CHUNKS)
            def _():
                pltpu.async_copy(il_hbm.at[cid], il_v.at[b], msem.at[b])
                pltpu.async_copy(
                    edge_hbm.at[pl.ds(cid * K, K), pl.ds(0, D)],
                    rows_v.at[b], rsem.at[b])

        def wait_meta(b):
            pltpu.make_async_copy(il_hbm.at[0], il_v.at[b], msem.at[b]).wait()

        def wait_rows(b):
            pltpu.make_async_copy(
                edge_hbm.at[pl.ds(0, K), pl.ds(0, D)],
                rows_v.at[b], rsem.at[b]).wait()

        def wait_scat(b):
            pltpu.make_async_copy(
                rows_v.at[b], acc.at[il_v.at[b, 0]], ssem.at[b]).wait()

        def consume(mm, b):
            cid = mm * NW + wid

            @pl.when(cid < HALF_CHUNKS)
            def _():
                wait_meta(b)
                wait_rows(b)
                for i in range(K // L):
                    vals = plsc.bitcast(
                        il_v[b, 1, pl.ds(i * L, L)], jnp.float32)
                    plsc.store_scatter(
                        rows_v.at[b], [lane + i * L, zero16], vals)
                pltpu.async_copy(
                    rows_v.at[b], acc.at[il_v.at[b, 0]], ssem.at[b],
                    add=True)

        for p in range(P):
            issue_gathers(p, p)

        ROUNDS = -(-M_MAX // NBUF)

        def body(r, carry):
            for b in range(NBUF):
                mm = r * NBUF + b
                consume(mm, b)
                kk = mm + P
                kb = (b + P) % NBUF
                kcid = kk * NW + wid

                @pl.when(kcid < HALF_CHUNKS)
                def _():
                    @pl.when(kk >= NBUF)
                    def _():
                        wait_scat(kb)
                    issue_gathers(kk, kb)
            return carry

        lax.fori_loop(0, ROUNDS, body, 0)

        for d in range(ROUNDS * NBUF - NBUF - P - 2, ROUNDS * NBUF):
            cid = d * NW + wid
            cid4 = (d + NBUF) * NW + wid

            @pl.when(jnp.logical_and(cid < HALF_CHUNKS,
                                     cid4 >= HALF_CHUNKS))
            def _():
                wait_scat(d % NBUF)

        plsc.subcore_barrier()

        pltpu.sync_copy(
            acc.at[pl.ds(s * SLAB, SLAB)],
            part_hbm.at[c, pl.ds(s * SLAB, SLAB)],
        )

    return k(edge_half, il3_half, zeros)


def _combine(vertex_attr, pa):
    def body(v_ref, pa_ref, o_ref):
        p = pa_ref[0] + pa_ref[1]
        o_ref[:, :D] = v_ref[...]
        o_ref[:, D:] = jnp.concatenate([p[:, 1:], p[:, :1]], axis=1)

    return pl.pallas_call(
        body,
        grid=(10,),
        in_specs=[
            pl.BlockSpec((1000, D), lambda i: (i, 0)),
            pl.BlockSpec((NC, 1000, D), lambda i: (0, i, 0)),
        ],
        out_specs=pl.BlockSpec((1000, 2 * D), lambda i: (i, 0)),
        out_shape=jax.ShapeDtypeStruct((N, 2 * D), jnp.float32),
    )(vertex_attr, pa)


def kernel(vertex_attr, edgeij_pair, edge_attr, g, batch):
    dst2 = edgeij_pair[1].reshape(CHUNKS, 1, K)
    last2 = lax.bitcast_convert_type(
        edge_attr[:, D].reshape(CHUNKS, 1, K), jnp.int32)
    il3 = jnp.concatenate([dst2, last2], axis=1)
    zeros = jnp.zeros((N_PAD, D), dtype=jnp.float32)
    pa = _sc_partial(edge_attr, il3, zeros)
    return _combine(vertex_attr, pa)


# speedup vs baseline: 1.3095x; 1.0979x over previous
import functools

import jax
import jax.numpy as jnp
from jax import lax
from jax.experimental import pallas as pl
from jax.experimental.pallas import tpu as pltpu
from jax.experimental.pallas import tpu_sc as plsc

N = 10000
E = 320000
D = 128

NC = 2
NS = 16
NW = NC * NS
L = 16

K = 128
CHUNKS = E // K
HALF_E = E
HALF_CHUNKS = HALF_E // K
M_MAX = -(-HALF_CHUNKS // NW)
NBUF = 3
P = 2

N_PAD = 10112
SLAB = N_PAD // NS


def _sc_partial(edge_half, il3_half, zeros):
    mesh = plsc.VectorSubcoreMesh(core_axis_name="c", subcore_axis_name="s")

    @functools.partial(
        pl.kernel,
        out_type=jax.ShapeDtypeStruct((NC, N_PAD, D), jnp.float32),
        mesh=mesh,
        scratch_types=[
            pltpu.VMEM((NBUF, 2, K), jnp.int32),
            pltpu.VMEM((NBUF, K, D), jnp.float32),
            pltpu.VMEM_SHARED((N_PAD, D), jnp.float32),
            pltpu.SemaphoreType.DMA((NBUF,)),
            pltpu.SemaphoreType.DMA((NBUF,)),
            pltpu.SemaphoreType.DMA((NBUF,)),
        ],
        compiler_params=pltpu.CompilerParams(needs_layout_passes=False),
    )
    def k(edge_hbm, il_hbm, zeros_hbm, part_hbm,
          il_v, rows_v, acc, msem, rsem, ssem):
        c = lax.axis_index("c")
        s = lax.axis_index("s")
        wid = s * NC + c
        lane = lax.iota(jnp.int32, L)
        zero16 = jnp.zeros((L,), jnp.int32)

        pltpu.sync_copy(
            zeros_hbm.at[pl.ds(s * SLAB, SLAB)],
            acc.at[pl.ds(s * SLAB, SLAB)],
        )
        plsc.subcore_barrier()

        def issue_gathers(mm, b):
            cid = mm * NW + wid

            @pl.when(cid < HALF_CHUNKS)
            def _():
                pltpu.async_copy(il_hbm.at[cid], il_v.at[b], msem.at[b])
                pltpu.async_copy(
                    edge_hbm.at[pl.ds(cid * K, K), pl.ds(0, D)],
                    rows_v.at[b], rsem.at[b])

        def wait_meta(b):
            pltpu.make_async_copy(il_hbm.at[0], il_v.at[b], msem.at[b]).wait()

        def wait_rows(b):
            pltpu.make_async_copy(
                edge_hbm.at[pl.ds(0, K), pl.ds(0, D)],
                rows_v.at[b], rsem.at[b]).wait()

        def wait_scat(b):
            pltpu.make_async_copy(
                rows_v.at[b], acc.at[il_v.at[b, 0]], ssem.at[b]).wait()

        def consume(mm, b):
            cid = mm * NW + wid

            @pl.when(cid < HALF_CHUNKS)
            def _():
                wait_meta(b)
                wait_rows(b)
                for i in range(K // L):
                    vals = plsc.bitcast(
                        il_v[b, 1, pl.ds(i * L, L)], jnp.float32)
                    plsc.store_scatter(
                        rows_v.at[b], [lane + i * L, zero16], vals)
                pltpu.async_copy(
                    rows_v.at[b], acc.at[il_v.at[b, 0]], ssem.at[b],
                    add=True)

        for p in range(P):
            issue_gathers(p, p)

        ROUNDS = -(-M_MAX // NBUF)

        def body(r, carry):
            for b in range(NBUF):
                mm = r * NBUF + b
                consume(mm, b)
                kk = mm + P
                kb = (b + P) % NBUF
                kcid = kk * NW + wid

                @pl.when(kcid < HALF_CHUNKS)
                def _():
                    @pl.when(kk >= NBUF)
                    def _():
                        wait_scat(kb)
                    issue_gathers(kk, kb)
            return carry

        lax.fori_loop(0, ROUNDS, body, 0)

        for d in range(ROUNDS * NBUF - NBUF - P - 2, ROUNDS * NBUF):
            cid = d * NW + wid
            cid4 = (d + NBUF) * NW + wid

            @pl.when(jnp.logical_and(cid < HALF_CHUNKS,
                                     cid4 >= HALF_CHUNKS))
            def _():
                wait_scat(d % NBUF)

        plsc.subcore_barrier()

        pltpu.sync_copy(
            acc.at[pl.ds(s * SLAB, SLAB)],
            part_hbm.at[c, pl.ds(s * SLAB, SLAB)],
        )

    return k(edge_half, il3_half, zeros)


def _combine(vertex_attr, pa):
    def body(v_ref, pa_ref, o_ref):
        p = pa_ref[0] + pa_ref[1]
        o_ref[:, :D] = v_ref[...]
        o_ref[:, D:] = jnp.concatenate([p[:, 1:], p[:, :1]], axis=1)

    return pl.pallas_call(
        body,
        grid=(10,),
        in_specs=[
            pl.BlockSpec((1000, D), lambda i: (i, 0)),
            pl.BlockSpec((NC, 1000, D), lambda i: (0, i, 0)),
        ],
        out_specs=pl.BlockSpec((1000, 2 * D), lambda i: (i, 0)),
        out_shape=jax.ShapeDtypeStruct((N, 2 * D), jnp.float32),
    )(vertex_attr, pa)


def kernel(vertex_attr, edgeij_pair, edge_attr, g, batch):
    dst2 = edgeij_pair[1].reshape(CHUNKS, 1, K)
    last2 = lax.bitcast_convert_type(
        edge_attr[:, D].reshape(CHUNKS, 1, K), jnp.int32)
    il3 = jnp.concatenate([dst2, last2], axis=1)
    zeros = jnp.zeros((N_PAD, D), dtype=jnp.float32)
    pa = _sc_partial(edge_attr, il3, zeros)
    return _combine(vertex_attr, pa)
